# Initial kernel scaffold; baseline (speedup 1.0000x reference)
import jax
import jax.numpy as jnp
from jax.experimental import pallas as pl


def kernel(input_ids_dna, input_ids_ideas, W_dna, W_ideas, W_pos, gamma, beta):
    raise NotImplementedError("write your pallas kernel here")


# speedup vs baseline: 4.7930x; 4.7930x over previous
import functools

import jax
import jax.numpy as jnp
from jax import lax
from jax.experimental import pallas as pl
from jax.experimental.pallas import tpu as pltpu
from jax.experimental.pallas import tpu_sc as plsc

_EPS = 1e-12

_NC = 2
_NS = 16
_NW = _NC * _NS

_T = 128


def _sc_gather_sum(dna, ideas, idxd3, idxi3, n_tokens, chunks, h):
    mesh = plsc.VectorSubcoreMesh(core_axis_name="c", subcore_axis_name="s")

    @functools.partial(
        pl.kernel,
        mesh=mesh,
        out_type=jax.ShapeDtypeStruct((n_tokens, h), jnp.float32),
        scratch_types=[
            pltpu.VMEM((chunks, _T), jnp.int32),
            pltpu.VMEM((chunks, _T), jnp.int32),
            pltpu.VMEM((_T, h), jnp.float32),
            pltpu.VMEM((_T, h), jnp.float32),
            pltpu.SemaphoreType.DMA,
            pltpu.SemaphoreType.DMA,
        ],
    )
    def k(dna_h, ideas_h, idxd_h, idxi_h, out_h, idxd_v, idxi_v,
          rows_d, rows_i, semd, semi):
        wid = lax.axis_index("s") * _NC + lax.axis_index("c")
        pltpu.sync_copy(idxd_h.at[wid], idxd_v)
        pltpu.sync_copy(idxi_h.at[wid], idxi_v)

        def chunk(c, carry):
            cpd = pltpu.async_copy(dna_h.at[idxd_v.at[c]], rows_d, semd)
            cpi = pltpu.async_copy(ideas_h.at[idxi_v.at[c]], rows_i, semi)
            cpd.wait()
            cpi.wait()

            def tok(t, carry2):
                for j in range(h // 16):
                    sl = pl.ds(j * 16, 16)
                    rows_d[t, sl] = rows_d[t, sl] + rows_i[t, sl]
                return carry2

            lax.fori_loop(0, _T, tok, 0)
            base = (wid * chunks + c) * _T
            pltpu.sync_copy(rows_d, out_h.at[pl.ds(base, _T)])
            return carry

        lax.fori_loop(0, chunks, chunk, 0)

    return k(dna, ideas, idxd3, idxi3)


def _tc_ln_body(x_ref, idd_ref, idi_ref, pos_ref, wd0_ref, wi0_ref,
                g_ref, b_ref, o_ref):
    x = x_ref[...]
    md = (idd_ref[...] == 0).astype(jnp.float32)[..., None]
    mi = (idi_ref[...] == 0).astype(jnp.float32)[..., None]
    x = (x
         - md * wd0_ref[0][None, None, :]
         - mi * wi0_ref[0][None, None, :]
         + pos_ref[...][None, :, :])
    mean = jnp.mean(x, axis=-1, keepdims=True)
    xc = x - mean
    var = jnp.mean(xc * xc, axis=-1, keepdims=True)
    o_ref[...] = (xc * lax.rsqrt(var + _EPS) * g_ref[0][None, None, :]
                  + b_ref[0][None, None, :])


def _tc_layernorm(x, ids_d, ids_i, pos, wd0, wi0, gamma2, beta2):
    b, s, h = x.shape
    bs = 16
    grid = (b // bs,)
    return pl.pallas_call(
        _tc_ln_body,
        grid=grid,
        in_specs=[
            pl.BlockSpec((bs, s, h), lambda i: (i, 0, 0)),
            pl.BlockSpec((bs, s), lambda i: (i, 0)),
            pl.BlockSpec((bs, s), lambda i: (i, 0)),
            pl.BlockSpec((s, h), lambda i: (0, 0)),
            pl.BlockSpec((1, h), lambda i: (0, 0)),
            pl.BlockSpec((1, h), lambda i: (0, 0)),
            pl.BlockSpec((1, h), lambda i: (0, 0)),
            pl.BlockSpec((1, h), lambda i: (0, 0)),
        ],
        out_specs=pl.BlockSpec((bs, s, h), lambda i: (i, 0, 0)),
        out_shape=jax.ShapeDtypeStruct((b, s, h), jnp.float32),
    )(x, ids_d, ids_i, pos, wd0, wi0, gamma2, beta2)


def kernel(input_ids_dna, input_ids_ideas, W_dna, W_ideas, W_pos, gamma, beta):
    b, s = input_ids_dna.shape
    v, h = W_dna.shape
    n_tokens = b * s
    chunks = n_tokens // (_NW * _T)

    idxd3 = input_ids_dna.reshape(_NW, chunks, _T)
    idxi3 = input_ids_ideas.reshape(_NW, chunks, _T)

    sums = _sc_gather_sum(W_dna, W_ideas, idxd3, idxi3, n_tokens, chunks, h)
    x = sums.reshape(b, s, h)

    return _tc_layernorm(
        x,
        input_ids_dna,
        input_ids_ideas,
        W_pos[:s],
        W_dna[0:1],
        W_ideas[0:1],
        gamma.reshape(1, h),
        beta.reshape(1, h),
    )


# speedup vs baseline: 6.2674x; 1.3076x over previous
import functools

import jax
import jax.numpy as jnp
from jax import lax
from jax.experimental import pallas as pl
from jax.experimental.pallas import tpu as pltpu
from jax.experimental.pallas import tpu_sc as plsc

_EPS = 1e-12

_NC = 2
_NS = 16
_NW = _NC * _NS

_T = 128


def _sc_gather_sum(dna, ideas, idxd3, idxi3, n_tokens, chunks, h):
    mesh = plsc.VectorSubcoreMesh(core_axis_name="c", subcore_axis_name="s")

    @functools.partial(
        pl.kernel,
        mesh=mesh,
        out_type=jax.ShapeDtypeStruct((n_tokens, h), jnp.float32),
        scratch_types=[
            pltpu.VMEM((chunks, _T), jnp.int32),
            pltpu.VMEM((chunks, _T), jnp.int32),
            pltpu.VMEM((_T, h), jnp.float32),
            pltpu.VMEM((_T, h), jnp.float32),
            pltpu.VMEM((_T, h), jnp.float32),
            pltpu.VMEM((_T, h), jnp.float32),
            pltpu.SemaphoreType.DMA,
            pltpu.SemaphoreType.DMA,
            pltpu.SemaphoreType.DMA,
            pltpu.SemaphoreType.DMA,
        ],
    )
    def k(dna_h, ideas_h, idxd_h, idxi_h, out_h, idxd_v, idxi_v,
          rows_d0, rows_i0, rows_d1, rows_i1, semd0, semi0, semd1, semi1):
        wid = lax.axis_index("s") * _NC + lax.axis_index("c")
        pltpu.sync_copy(idxd_h.at[wid], idxd_v)
        pltpu.sync_copy(idxi_h.at[wid], idxi_v)

        bufs = ((rows_d0, rows_i0, semd0, semi0),
                (rows_d1, rows_i1, semd1, semi1))

        def start(c, b):
            rows_d, rows_i, semd, semi = bufs[b]
            pltpu.async_copy(dna_h.at[idxd_v.at[c]], rows_d, semd)
            pltpu.async_copy(ideas_h.at[idxi_v.at[c]], rows_i, semi)

        def finish(c, b):
            rows_d, rows_i, semd, semi = bufs[b]
            pltpu.make_async_copy(dna_h.at[idxd_v.at[c]], rows_d, semd).wait()
            pltpu.make_async_copy(ideas_h.at[idxi_v.at[c]], rows_i, semi).wait()

            def tok(t, carry2):
                for u in range(2):
                    for j in range(h // 16):
                        sl = pl.ds(j * 16, 16)
                        rows_d[2 * t + u, sl] = (rows_d[2 * t + u, sl]
                                                 + rows_i[2 * t + u, sl])
                return carry2

            lax.fori_loop(0, _T // 2, tok, 0)
            base = (wid * chunks + c) * _T
            pltpu.sync_copy(rows_d, out_h.at[pl.ds(base, _T)])

        start(0, 0)

        def pair(g, carry):
            for b in range(2):
                c = 2 * g + b

                @pl.when(c + 1 < chunks)
                def _():
                    start(c + 1, 1 - b)

                finish(c, b)
            return carry

        lax.fori_loop(0, chunks // 2, pair, 0)

    return k(dna, ideas, idxd3, idxi3)


def _tc_ln_body(x_ref, idd_ref, idi_ref, pos_ref, wd0_ref, wi0_ref,
                g_ref, b_ref, o_ref):
    x = x_ref[...]
    md = (idd_ref[...] == 0).astype(jnp.float32)[..., None]
    mi = (idi_ref[...] == 0).astype(jnp.float32)[..., None]
    x = (x
         - md * wd0_ref[0][None, None, :]
         - mi * wi0_ref[0][None, None, :]
         + pos_ref[...][None, :, :])
    mean = jnp.mean(x, axis=-1, keepdims=True)
    xc = x - mean
    var = jnp.mean(xc * xc, axis=-1, keepdims=True)
    o_ref[...] = (xc * lax.rsqrt(var + _EPS) * g_ref[0][None, None, :]
                  + b_ref[0][None, None, :])


def _tc_layernorm(x, ids_d, ids_i, pos, wd0, wi0, gamma2, beta2):
    b, s, h = x.shape
    bs = 16
    grid = (b // bs,)
    return pl.pallas_call(
        _tc_ln_body,
        grid=grid,
        in_specs=[
            pl.BlockSpec((bs, s, h), lambda i: (i, 0, 0)),
            pl.BlockSpec((bs, s), lambda i: (i, 0)),
            pl.BlockSpec((bs, s), lambda i: (i, 0)),
            pl.BlockSpec((s, h), lambda i: (0, 0)),
            pl.BlockSpec((1, h), lambda i: (0, 0)),
            pl.BlockSpec((1, h), lambda i: (0, 0)),
            pl.BlockSpec((1, h), lambda i: (0, 0)),
            pl.BlockSpec((1, h), lambda i: (0, 0)),
        ],
        out_specs=pl.BlockSpec((bs, s, h), lambda i: (i, 0, 0)),
        out_shape=jax.ShapeDtypeStruct((b, s, h), jnp.float32),
    )(x, ids_d, ids_i, pos, wd0, wi0, gamma2, beta2)


def kernel(input_ids_dna, input_ids_ideas, W_dna, W_ideas, W_pos, gamma, beta):
    b, s = input_ids_dna.shape
    v, h = W_dna.shape
    n_tokens = b * s
    chunks = n_tokens // (_NW * _T)

    idxd3 = input_ids_dna.reshape(_NW, chunks, _T)
    idxi3 = input_ids_ideas.reshape(_NW, chunks, _T)

    sums = _sc_gather_sum(W_dna, W_ideas, idxd3, idxi3, n_tokens, chunks, h)
    x = sums.reshape(b, s, h)

    return _tc_layernorm(
        x,
        input_ids_dna,
        input_ids_ideas,
        W_pos[:s],
        W_dna[0:1],
        W_ideas[0:1],
        gamma.reshape(1, h),
        beta.reshape(1, h),
    )
